# baseline (speedup 1.0000x reference)
import jax
import jax.numpy as jnp
from jax import lax
from jax.experimental import pallas as pl
from jax.experimental.pallas import tpu as pltpu
from jax.experimental.pallas import tpu_sc as plsc

BATCH = 16384
HIST = 50
EMB_DIM = 64

NUM_CORES = 2
NUM_SUBCORES = 16
NW = NUM_CORES * NUM_SUBCORES
B_PER_W = BATCH // NW
NBUF = 3


def _body(xt_hbm, table_hbm, out_hbm, iv_all, rv_a, rv_b, rv_c,
          sem_i, sem_ga, sem_gb, sem_gc, sem_wa, sem_wb, sem_wc):
    wid = lax.axis_index("s") * NUM_CORES + lax.axis_index("c")
    b0 = wid * B_PER_W

    pltpu.async_copy(xt_hbm.at[:, pl.ds(b0, B_PER_W)], iv_all, sem_i).wait()

    bufs = (rv_a, rv_b, rv_c)
    gsems = (sem_ga, sem_gb, sem_gc)
    wsems = (sem_wa, sem_wb, sem_wc)

    def gather(h, p):
        pltpu.async_copy(table_hbm.at[iv_all.at[h]], bufs[p], gsems[p])

    def gwait(h, p):
        pltpu.make_async_copy(
            table_hbm.at[iv_all.at[h]], bufs[p], gsems[p]).wait()

    def wcopy(h, p):
        return pltpu.make_async_copy(
            bufs[p], out_hbm.at[h, pl.ds(b0, B_PER_W)], wsems[p])

    gather(0, 0)
    gather(1, 1)
    for h in range(HIST):
        p = h % NBUF
        gwait(h, p)
        wcopy(h, p).start()
        if h + 2 < HIST:
            q = (h + 2) % NBUF
            if h >= 1:
                wcopy(h - 1, q).wait()
            gather(h + 2, q)
    wcopy(HIST - 2, (HIST - 2) % NBUF).wait()
    wcopy(HIST - 1, (HIST - 1) % NBUF).wait()


@jax.jit
def _gather(x_t, table):
    mesh = plsc.VectorSubcoreMesh(core_axis_name="c", subcore_axis_name="s")
    k = pl.kernel(
        _body,
        out_type=jax.ShapeDtypeStruct((HIST, BATCH, EMB_DIM), jnp.float32),
        mesh=mesh,
        compiler_params=pltpu.CompilerParams(
            use_tc_tiling_on_sc=False, needs_layout_passes=False),
        scratch_types=[
            pltpu.VMEM((HIST, B_PER_W), jnp.int32),
            pltpu.VMEM((B_PER_W, EMB_DIM), jnp.float32),
            pltpu.VMEM((B_PER_W, EMB_DIM), jnp.float32),
            pltpu.VMEM((B_PER_W, EMB_DIM), jnp.float32),
            pltpu.SemaphoreType.DMA,
            pltpu.SemaphoreType.DMA,
            pltpu.SemaphoreType.DMA,
            pltpu.SemaphoreType.DMA,
            pltpu.SemaphoreType.DMA,
            pltpu.SemaphoreType.DMA,
            pltpu.SemaphoreType.DMA,
        ],
    )
    return k(x_t, table)


def kernel(x, table):
    x_t = x.T.astype(jnp.int32)
    out3 = _gather(x_t, table)
    return out3.transpose(1, 0, 2)
